# baseline (speedup 1.0000x reference)
import functools

import jax
import jax.numpy as jnp
from jax import lax
from jax.experimental import pallas as pl
from jax.experimental.pallas import tpu as pltpu
from jax.experimental.pallas import tpu_sc as plsc

N = 10000
NP = 10240
E = 320000
D = 128
NC = 2
NS = 16
NW = NC * NS
ET = E // NS
EW = E // NW
CH = 5
CS = EW // CH
NT = NP // NS

_mesh = plsc.VectorSubcoreMesh(core_axis_name="c", subcore_axis_name="s")


def _nrsqrt(d):
    magic = jnp.full((16,), 0x5F3759DF, jnp.int32)
    one = jnp.full((16,), 1, jnp.int32)
    half = jnp.full((16,), 0.5, jnp.float32)
    three_half = jnp.full((16,), 1.5, jnp.float32)
    y = lax.bitcast_convert_type(
        magic - lax.shift_right_logical(
            lax.bitcast_convert_type(d, jnp.int32), one),
        jnp.float32)
    for _ in range(3):
        y = y * (three_half - half * d * y * y)
    return y


@functools.partial(
    pl.kernel,
    out_type=(
        jax.ShapeDtypeStruct((NC, NP), jnp.float32),
        jax.ShapeDtypeStruct((NC, NP), jnp.float32),
    ),
    mesh=_mesh,
    scratch_types=[
        pltpu.VMEM((ET,), jnp.int32),
        pltpu.VMEM((EW,), jnp.int32),
        [pltpu.VMEM((CS,), jnp.int32) for _ in range(CH)],
        [pltpu.VMEM((CS,), jnp.int32) for _ in range(CH)],
        [pltpu.VMEM((CS,), jnp.float32) for _ in range(CH)],
        pltpu.VMEM((ET,), jnp.float32),
        pltpu.VMEM((NT,), jnp.float32),
        pltpu.VMEM((NT,), jnp.float32),
        pltpu.VMEM((NT,), jnp.float32),
        pltpu.VMEM_SHARED((NP,), jnp.float32),
        pltpu.VMEM_SHARED((NP,), jnp.float32),
        pltpu.VMEM_SHARED((NP,), jnp.float32),
        pltpu.VMEM_SHARED((NP,), jnp.float32),
        pltpu.SemaphoreType.DMA,
        pltpu.SemaphoreType.DMA,
        pltpu.SemaphoreType.DMA,
    ],
)
def _k_sc(src_h, dst_h, src_g, dst_g, t_hbm, zeros_hbm, ones_hbm, agg_out,
          ideg_out,
          sh_v, dh_v, sg, dg, vals, ones_v, deg_v, t_v, s_v,
          od_acc, id_acc, agg_acc, s_sh, hsem, gsem, ssem):
    cid = lax.axis_index("c")
    sid = lax.axis_index("s")
    wid = 2 * sid + cid

    @pl.when(sid == 0)
    def _():
        pltpu.sync_copy(zeros_hbm, od_acc)

    @pl.when(sid == 1)
    def _():
        pltpu.sync_copy(zeros_hbm, id_acc)

    @pl.when(sid == 2)
    def _():
        pltpu.sync_copy(zeros_hbm, agg_acc)

    pltpu.sync_copy(src_h.at[sid], sh_v)
    pltpu.sync_copy(dst_h.at[wid], dh_v)
    for j in range(CH):
        pltpu.sync_copy(src_g.at[wid * CH + j], sg[j])
        pltpu.sync_copy(dst_g.at[wid * CH + j], dg[j])
    pltpu.sync_copy(ones_hbm, ones_v)
    plsc.subcore_barrier()

    h1 = pltpu.async_copy(ones_v, od_acc.at[sh_v], hsem, add=True)
    h2 = pltpu.async_copy(ones_v.at[pl.ds(0, EW)], id_acc.at[dh_v], hsem,
                          add=True)
    h1.wait()
    h2.wait()
    plsc.subcore_barrier()

    pltpu.sync_copy(od_acc.at[pl.ds(sid * NT, NT)], deg_v)
    pltpu.sync_copy(t_hbm.at[pl.ds(sid * NT, NT)], t_v)
    for i in range(NT // 16):
        ix = pl.ds(i * 16, 16)
        s_v[ix] = _nrsqrt(jnp.maximum(deg_v[ix], 1.0)) * t_v[ix]
    pltpu.sync_copy(s_v, s_sh.at[pl.ds(sid * NT, NT)])
    plsc.subcore_barrier()

    gathers = [None] * CH
    gathers[0] = pltpu.async_copy(s_sh.at[sg[0]], vals[0], gsem)
    scatter = None
    for j in range(CH):
        gathers[j].wait()
        if j + 1 < CH:
            gathers[j + 1] = pltpu.async_copy(s_sh.at[sg[j + 1]], vals[j + 1],
                                              gsem)
        if scatter is not None:
            scatter.wait()
        scatter = pltpu.async_copy(vals[j], agg_acc.at[dg[j]], ssem, add=True)
    scatter.wait()
    plsc.subcore_barrier()

    @pl.when(sid == 0)
    def _():
        pltpu.sync_copy(agg_acc, agg_out.at[cid])

    @pl.when(sid == 1)
    def _():
        pltpu.sync_copy(id_acc, ideg_out.at[cid])


def _kt_body(x_ref, wg_ref, wc_ref, t_ref):
    w = jnp.dot(wg_ref[...], wc_ref[...], preferred_element_type=jnp.float32)
    t_ref[...] = jnp.dot(x_ref[...], w,
                         preferred_element_type=jnp.float32).reshape(N)


_kt_matvec = pl.pallas_call(
    _kt_body,
    out_shape=jax.ShapeDtypeStruct((N,), jnp.float32),
)


def _k3_body(agg_ref, id_ref, bg_ref, wc_ref, bc_ref, y_ref):
    const = jnp.sum(bg_ref[...] * wc_ref[...]) + bc_ref[0, 0]
    nd = lax.rsqrt(jnp.maximum(id_ref[0] + id_ref[1], 1.0))
    y = nd * (agg_ref[0] + agg_ref[1]) + const
    y_ref[...] = y.reshape(N, 1)


_k3_combine = pl.pallas_call(
    _k3_body,
    out_shape=jax.ShapeDtypeStruct((N, 1), jnp.float32),
)


def kernel(x, edge_index, W_gc, b_gc, W_cls, b_cls):
    t = _kt_matvec(x, W_gc, W_cls)
    t_pad = jnp.concatenate([t, jnp.zeros((NP - N,), jnp.float32)])
    src_h = edge_index[0].reshape(NS, ET)
    dst_h = edge_index[1].reshape(NW, EW)
    src_g = edge_index[0].reshape(NW * CH, CS)
    dst_g = edge_index[1].reshape(NW * CH, CS)
    zeros = jnp.zeros((NP,), jnp.float32)
    ones = jnp.ones((ET,), jnp.float32)
    agg, ideg = _k_sc(src_h, dst_h, src_g, dst_g, t_pad, zeros, ones)
    y = _k3_combine(agg[:, :N], ideg[:, :N],
                    b_gc.reshape(1, D), W_cls.reshape(1, D),
                    b_cls.reshape(1, 1))
    return y


# speedup vs baseline: 1.0020x; 1.0020x over previous
import functools

import jax
import jax.numpy as jnp
from jax import lax
from jax.experimental import pallas as pl
from jax.experimental.pallas import tpu as pltpu
from jax.experimental.pallas import tpu_sc as plsc

N = 10000
NP = 10240
E = 320000
D = 128
NC = 2
NS = 16
NW = NC * NS
ET = E // NS
EW = E // NW
CH = 5
CS = EW // CH
NT = NP // NS

_mesh = plsc.VectorSubcoreMesh(core_axis_name="c", subcore_axis_name="s")


def _nrsqrt(d):
    magic = jnp.full((16,), 0x5F3759DF, jnp.int32)
    one = jnp.full((16,), 1, jnp.int32)
    half = jnp.full((16,), 0.5, jnp.float32)
    three_half = jnp.full((16,), 1.5, jnp.float32)
    y = lax.bitcast_convert_type(
        magic - lax.shift_right_logical(
            lax.bitcast_convert_type(d, jnp.int32), one),
        jnp.float32)
    for _ in range(3):
        y = y * (three_half - half * d * y * y)
    return y


@functools.partial(
    pl.kernel,
    out_type=(
        jax.ShapeDtypeStruct((NC, NP), jnp.float32),
        jax.ShapeDtypeStruct((NC, NP), jnp.float32),
    ),
    mesh=_mesh,
    scratch_types=[
        pltpu.VMEM((ET,), jnp.int32),
        pltpu.VMEM((EW,), jnp.int32),
        [pltpu.VMEM((CS,), jnp.int32) for _ in range(CH)],
        [pltpu.VMEM((CS,), jnp.int32) for _ in range(CH)],
        [pltpu.VMEM((CS,), jnp.float32) for _ in range(CH)],
        pltpu.VMEM((ET,), jnp.float32),
        pltpu.VMEM((NT,), jnp.float32),
        pltpu.VMEM((NT,), jnp.float32),
        pltpu.VMEM((NT,), jnp.float32),
        pltpu.VMEM_SHARED((NP,), jnp.float32),
        pltpu.VMEM_SHARED((NP,), jnp.float32),
        pltpu.VMEM_SHARED((NP,), jnp.float32),
        pltpu.VMEM_SHARED((NP,), jnp.float32),
        pltpu.SemaphoreType.DMA,
        pltpu.SemaphoreType.DMA,
        pltpu.SemaphoreType.DMA,
    ],
)
def _k_sc(src_h, dst_h, src_g, dst_g, t_hbm, zeros_hbm, ones_hbm, agg_out,
          ideg_out,
          sh_v, dh_v, sg, dg, vals, ones_v, deg_v, t_v, s_v,
          od_acc, id_acc, agg_acc, s_sh, hsem, gsem, ssem):
    cid = lax.axis_index("c")
    sid = lax.axis_index("s")
    wid = 2 * sid + cid

    @pl.when(sid == 0)
    def _():
        pltpu.sync_copy(zeros_hbm, od_acc)

    @pl.when(sid == 1)
    def _():
        pltpu.sync_copy(zeros_hbm, id_acc)

    @pl.when(sid == 2)
    def _():
        pltpu.sync_copy(zeros_hbm, agg_acc)

    pltpu.sync_copy(src_h.at[sid], sh_v)
    pltpu.sync_copy(dst_h.at[wid], dh_v)
    for j in range(CH):
        pltpu.sync_copy(src_g.at[wid * CH + j], sg[j])
        pltpu.sync_copy(dst_g.at[wid * CH + j], dg[j])
    pltpu.sync_copy(ones_hbm, ones_v)
    plsc.subcore_barrier()

    h1 = pltpu.async_copy(ones_v, od_acc.at[sh_v], hsem, add=True)
    h2 = pltpu.async_copy(ones_v.at[pl.ds(0, EW)], id_acc.at[dh_v], hsem,
                          add=True)
    h1.wait()
    h2.wait()
    plsc.subcore_barrier()

    pltpu.sync_copy(od_acc.at[pl.ds(sid * NT, NT)], deg_v)
    pltpu.sync_copy(t_hbm.at[pl.ds(sid * NT, NT)], t_v)
    def _norm_step(i, carry):
        ix = pl.ds(i * 16, 16)
        s_v[ix] = _nrsqrt(jnp.maximum(deg_v[ix], 1.0)) * t_v[ix]
        return carry

    lax.fori_loop(0, NT // 16, _norm_step, 0)
    pltpu.sync_copy(s_v, s_sh.at[pl.ds(sid * NT, NT)])
    plsc.subcore_barrier()

    gathers = [None] * CH
    gathers[0] = pltpu.async_copy(s_sh.at[sg[0]], vals[0], gsem)
    scatter = None
    for j in range(CH):
        gathers[j].wait()
        if j + 1 < CH:
            gathers[j + 1] = pltpu.async_copy(s_sh.at[sg[j + 1]], vals[j + 1],
                                              gsem)
        if scatter is not None:
            scatter.wait()
        scatter = pltpu.async_copy(vals[j], agg_acc.at[dg[j]], ssem, add=True)
    scatter.wait()
    plsc.subcore_barrier()

    @pl.when(sid == 0)
    def _():
        pltpu.sync_copy(agg_acc, agg_out.at[cid])

    @pl.when(sid == 1)
    def _():
        pltpu.sync_copy(id_acc, ideg_out.at[cid])


def _kt_body(x_ref, wg_ref, wc_ref, t_ref):
    w = jnp.dot(wg_ref[...], wc_ref[...], preferred_element_type=jnp.float32)
    t_ref[...] = jnp.dot(x_ref[...], w,
                         preferred_element_type=jnp.float32).reshape(N)


_kt_matvec = pl.pallas_call(
    _kt_body,
    out_shape=jax.ShapeDtypeStruct((N,), jnp.float32),
)


def _k3_body(agg_ref, id_ref, bg_ref, wc_ref, bc_ref, y_ref):
    const = jnp.sum(bg_ref[...] * wc_ref[...]) + bc_ref[0, 0]
    nd = lax.rsqrt(jnp.maximum(id_ref[0] + id_ref[1], 1.0))
    y = nd * (agg_ref[0] + agg_ref[1]) + const
    y_ref[...] = y.reshape(N, 1)


_k3_combine = pl.pallas_call(
    _k3_body,
    out_shape=jax.ShapeDtypeStruct((N, 1), jnp.float32),
)


def kernel(x, edge_index, W_gc, b_gc, W_cls, b_cls):
    t = _kt_matvec(x, W_gc, W_cls)
    t_pad = jnp.concatenate([t, jnp.zeros((NP - N,), jnp.float32)])
    src_h = edge_index[0].reshape(NS, ET)
    dst_h = edge_index[1].reshape(NW, EW)
    src_g = edge_index[0].reshape(NW * CH, CS)
    dst_g = edge_index[1].reshape(NW * CH, CS)
    zeros = jnp.zeros((NP,), jnp.float32)
    ones = jnp.ones((ET,), jnp.float32)
    agg, ideg = _k_sc(src_h, dst_h, src_g, dst_g, t_pad, zeros, ones)
    y = _k3_combine(agg[:, :N], ideg[:, :N],
                    b_gc.reshape(1, D), W_cls.reshape(1, D),
                    b_cls.reshape(1, 1))
    return y


# speedup vs baseline: 1.0910x; 1.0889x over previous
import functools

import jax
import jax.numpy as jnp
from jax import lax
from jax.experimental import pallas as pl
from jax.experimental.pallas import tpu as pltpu
from jax.experimental.pallas import tpu_sc as plsc

N = 10000
NP = 10240
E = 320000
D = 128
NC = 2
NS = 16
NW = NC * NS
ET = E // NS
EW = E // NW
CH = 5
CS = EW // CH
NT = NP // NS

_mesh = plsc.VectorSubcoreMesh(core_axis_name="c", subcore_axis_name="s")


def _nrsqrt(d):
    magic = jnp.full((16,), 0x5F3759DF, jnp.int32)
    one = jnp.full((16,), 1, jnp.int32)
    half = jnp.full((16,), 0.5, jnp.float32)
    three_half = jnp.full((16,), 1.5, jnp.float32)
    y = lax.bitcast_convert_type(
        magic - lax.shift_right_logical(
            lax.bitcast_convert_type(d, jnp.int32), one),
        jnp.float32)
    for _ in range(3):
        y = y * (three_half - half * d * y * y)
    return y


@functools.partial(
    pl.kernel,
    out_type=(
        jax.ShapeDtypeStruct((NC, NP), jnp.float32),
        jax.ShapeDtypeStruct((NC, NP), jnp.float32),
    ),
    mesh=_mesh,
    scratch_types=[
        pltpu.VMEM((ET,), jnp.int32),
        [pltpu.VMEM((CS,), jnp.int32) for _ in range(CH)],
        pltpu.VMEM((EW,), jnp.float32),
        pltpu.VMEM((ET,), jnp.float32),
        pltpu.VMEM((NT,), jnp.float32),
        pltpu.VMEM((NT,), jnp.float32),
        pltpu.VMEM((NT,), jnp.float32),
        pltpu.VMEM_SHARED((NP,), jnp.float32),
        pltpu.VMEM_SHARED((NP,), jnp.float32),
        pltpu.VMEM_SHARED((NP,), jnp.float32),
        pltpu.VMEM_SHARED((NP,), jnp.float32),
        pltpu.SemaphoreType.DMA,
        pltpu.SemaphoreType.DMA,
        pltpu.SemaphoreType.DMA,
    ],
)
def _k_sc(src_h, dst_g, t_hbm, zeros_hbm, ones_hbm, agg_out, ideg_out,
          sh_v, dg, vals, ones_v, deg_v, t_v, s_v,
          od_acc, id_acc, agg_acc, s_sh, hsem, gsem, ssem):
    cid = lax.axis_index("c")
    sid = lax.axis_index("s")
    wid = 2 * sid + cid

    @pl.when(sid == 0)
    def _():
        pltpu.sync_copy(zeros_hbm, od_acc)

    @pl.when(sid == 1)
    def _():
        pltpu.sync_copy(zeros_hbm, id_acc)

    @pl.when(sid == 2)
    def _():
        pltpu.sync_copy(zeros_hbm, agg_acc)

    pltpu.sync_copy(src_h.at[sid], sh_v)
    for j in range(CH):
        pltpu.sync_copy(dst_g.at[wid * CH + j], dg[j])
    pltpu.sync_copy(ones_hbm, ones_v)
    plsc.subcore_barrier()

    descs = [pltpu.async_copy(ones_v, od_acc.at[sh_v], hsem, add=True)]
    for j in range(CH):
        descs.append(pltpu.async_copy(ones_v.at[pl.ds(0, CS)],
                                      id_acc.at[dg[j]], hsem, add=True))
    for _d in descs:
        _d.wait()
    plsc.subcore_barrier()

    pltpu.sync_copy(od_acc.at[pl.ds(sid * NT, NT)], deg_v)
    pltpu.sync_copy(t_hbm.at[pl.ds(sid * NT, NT)], t_v)
    def _norm_step(i, carry):
        ix = pl.ds(i * 16, 16)
        s_v[ix] = _nrsqrt(jnp.maximum(deg_v[ix], 1.0)) * t_v[ix]
        return carry

    lax.fori_loop(0, NT // 16, _norm_step, 0)
    pltpu.sync_copy(s_v, s_sh.at[pl.ds(sid * NT, NT)])
    plsc.subcore_barrier()

    def _sg_ix(j):
        return sh_v.at[pl.ds(cid * EW + j * CS, CS)]

    def _vals_ix(j):
        return vals.at[pl.ds(j * CS, CS)]

    gathers = [None] * CH
    gathers[0] = pltpu.async_copy(s_sh.at[_sg_ix(0)], _vals_ix(0), gsem)
    scatter = None
    for j in range(CH):
        gathers[j].wait()
        if j + 1 < CH:
            gathers[j + 1] = pltpu.async_copy(s_sh.at[_sg_ix(j + 1)],
                                              _vals_ix(j + 1), gsem)
        if scatter is not None:
            scatter.wait()
        scatter = pltpu.async_copy(_vals_ix(j), agg_acc.at[dg[j]], ssem,
                                   add=True)
    scatter.wait()
    plsc.subcore_barrier()

    @pl.when(sid == 0)
    def _():
        pltpu.sync_copy(agg_acc, agg_out.at[cid])

    @pl.when(sid == 1)
    def _():
        pltpu.sync_copy(id_acc, ideg_out.at[cid])


def _kt_body(x_ref, wg_ref, wc_ref, t_ref):
    w = jnp.dot(wg_ref[...], wc_ref[...], preferred_element_type=jnp.float32)
    t_ref[...] = jnp.dot(x_ref[...], w,
                         preferred_element_type=jnp.float32).reshape(N)


_kt_matvec = pl.pallas_call(
    _kt_body,
    out_shape=jax.ShapeDtypeStruct((N,), jnp.float32),
)


def _k3_body(agg_ref, id_ref, bg_ref, wc_ref, bc_ref, y_ref):
    const = jnp.sum(bg_ref[...] * wc_ref[...]) + bc_ref[0, 0]
    nd = lax.rsqrt(jnp.maximum(id_ref[0] + id_ref[1], 1.0))
    y = nd * (agg_ref[0] + agg_ref[1]) + const
    y_ref[...] = y.reshape(N, 1)


_k3_combine = pl.pallas_call(
    _k3_body,
    out_shape=jax.ShapeDtypeStruct((N, 1), jnp.float32),
)


def kernel(x, edge_index, W_gc, b_gc, W_cls, b_cls):
    t = _kt_matvec(x, W_gc, W_cls)
    t_pad = jnp.concatenate([t, jnp.zeros((NP - N,), jnp.float32)])
    src_h = edge_index[0].reshape(NS, ET)
    dst_g = edge_index[1].reshape(NW * CH, CS)
    zeros = jnp.zeros((NP,), jnp.float32)
    ones = jnp.ones((ET,), jnp.float32)
    agg, ideg = _k_sc(src_h, dst_g, t_pad, zeros, ones)
    y = _k3_combine(agg[:, :N], ideg[:, :N],
                    b_gc.reshape(1, D), W_cls.reshape(1, D),
                    b_cls.reshape(1, 1))
    return y


# speedup vs baseline: 1.2381x; 1.1349x over previous
import functools

import jax
import jax.numpy as jnp
from jax import lax
from jax.experimental import pallas as pl
from jax.experimental.pallas import tpu as pltpu
from jax.experimental.pallas import tpu_sc as plsc

N = 10000
E = 320000
D = 128
NC = 2
NS = 16
NW = NC * NS
EW = E // NW
ET = E // NS
K0_CH = 2
K2_CH = 5
K2_CS = EW // K2_CH

_mesh = plsc.VectorSubcoreMesh(core_axis_name="c", subcore_axis_name="s")


@functools.partial(
    pl.kernel,
    out_type=jax.ShapeDtypeStruct((NC, N), jnp.float32),
    mesh=_mesh,
    scratch_types=[
        [pltpu.VMEM((ET // K0_CH,), jnp.int32) for _ in range(K0_CH)],
        pltpu.VMEM((ET // K0_CH,), jnp.float32),
        pltpu.VMEM_SHARED((N,), jnp.float32),
        pltpu.SemaphoreType.DMA,
    ],
)
def _k0_degrees(idx_hbm, zeros_hbm, ones_hbm, deg_out, idx_v, ones_v, acc, sem):
    cid = lax.axis_index("c")
    sid = lax.axis_index("s")

    @pl.when(sid == 0)
    def _():
        pltpu.sync_copy(zeros_hbm, acc)

    for j in range(K0_CH):
        pltpu.sync_copy(idx_hbm.at[(cid * NS + sid) * K0_CH + j], idx_v[j])
    pltpu.sync_copy(ones_hbm, ones_v)
    plsc.subcore_barrier()

    descs = [
        pltpu.async_copy(ones_v, acc.at[idx_v[j]], sem, add=True)
        for j in range(K0_CH)
    ]
    for d in descs:
        d.wait()
    plsc.subcore_barrier()

    @pl.when(sid == 0)
    def _():
        pltpu.sync_copy(acc, deg_out.at[cid])


def _k1_body(x_ref, wg_ref, wc_ref, deg_ref, s_ref, nd_ref):
    w = jnp.dot(wg_ref[...], wc_ref[...], preferred_element_type=jnp.float32)
    t = jnp.dot(x_ref[...], w, preferred_element_type=jnp.float32)
    s_ref[...] = lax.rsqrt(jnp.maximum(deg_ref[0], 1.0)) * t.reshape(N)
    nd_ref[...] = lax.rsqrt(jnp.maximum(deg_ref[1], 1.0))


_k1_scale = pl.pallas_call(
    _k1_body,
    out_shape=(
        jax.ShapeDtypeStruct((N,), jnp.float32),
        jax.ShapeDtypeStruct((N,), jnp.float32),
    ),
)


@functools.partial(
    pl.kernel,
    out_type=jax.ShapeDtypeStruct((NC, N), jnp.float32),
    mesh=_mesh,
    scratch_types=[
        [pltpu.VMEM((K2_CS,), jnp.int32) for _ in range(K2_CH)],
        [pltpu.VMEM((K2_CS,), jnp.int32) for _ in range(K2_CH)],
        [pltpu.VMEM((K2_CS,), jnp.float32) for _ in range(K2_CH)],
        pltpu.VMEM_SHARED((N,), jnp.float32),
        pltpu.VMEM_SHARED((N,), jnp.float32),
        pltpu.SemaphoreType.DMA,
        pltpu.SemaphoreType.DMA,
    ],
)
def _k2_scatter(src_hbm, dst_hbm, s_hbm, zeros_hbm, parts_out,
                idx_s, idx_d, vals, acc, s_sh, gsem, ssem):
    cid = lax.axis_index("c")
    sid = lax.axis_index("s")
    wid = cid * NS + sid

    @pl.when(sid == 0)
    def _():
        pltpu.sync_copy(zeros_hbm, acc)

    @pl.when(sid == 1)
    def _():
        pltpu.sync_copy(s_hbm, s_sh)

    for j in range(K2_CH):
        pltpu.sync_copy(src_hbm.at[wid * K2_CH + j], idx_s[j])
        pltpu.sync_copy(dst_hbm.at[wid * K2_CH + j], idx_d[j])
    plsc.subcore_barrier()

    gathers = [None] * K2_CH
    gathers[0] = pltpu.async_copy(s_sh.at[idx_s[0]], vals[0], gsem)
    scatter = None
    for j in range(K2_CH):
        gathers[j].wait()
        if j + 1 < K2_CH:
            gathers[j + 1] = pltpu.async_copy(
                s_sh.at[idx_s[j + 1]], vals[j + 1], gsem)
        if scatter is not None:
            scatter.wait()
        scatter = pltpu.async_copy(
            vals[j], acc.at[idx_d[j]], ssem, add=True)
    scatter.wait()
    plsc.subcore_barrier()

    @pl.when(sid == 0)
    def _():
        pltpu.sync_copy(acc, parts_out.at[cid])


def _k3_body(p_ref, nd_ref, bg_ref, wc_ref, bc_ref, y_ref):
    const = jnp.sum(bg_ref[...] * wc_ref[...]) + bc_ref[0, 0]
    y = nd_ref[...] * (p_ref[0] + p_ref[1]) + const
    y_ref[...] = y.reshape(N, 1)


_k3_combine = pl.pallas_call(
    _k3_body,
    out_shape=jax.ShapeDtypeStruct((N, 1), jnp.float32),
)


def kernel(x, edge_index, W_gc, b_gc, W_cls, b_cls):
    idx_all = edge_index.reshape(NW * K0_CH, ET // K0_CH)
    zeros = jnp.zeros((N,), jnp.float32)
    ones = jnp.ones((ET // K0_CH,), jnp.float32)
    deg = _k0_degrees(idx_all, zeros, ones)

    s, nd = _k1_scale(x, W_gc, W_cls, deg)

    src_b = edge_index[0].reshape(NW * K2_CH, K2_CS)
    dst_b = edge_index[1].reshape(NW * K2_CH, K2_CS)
    parts = _k2_scatter(src_b, dst_b, s, zeros)

    y = _k3_combine(parts, nd,
                    b_gc.reshape(1, D), W_cls.reshape(1, D),
                    b_cls.reshape(1, 1))
    return y


# speedup vs baseline: 1.2813x; 1.0349x over previous
import functools

import jax
import jax.numpy as jnp
from jax import lax
from jax.experimental import pallas as pl
from jax.experimental.pallas import tpu as pltpu
from jax.experimental.pallas import tpu_sc as plsc

N = 10000
E = 320000
D = 128
NC = 2
NS = 16
NW = NC * NS
EW = E // NW
ET = E // NS
K0_CH = 2
K2_CH = 5
K2_CS = EW // K2_CH

_mesh = plsc.VectorSubcoreMesh(core_axis_name="c", subcore_axis_name="s")


@functools.partial(
    pl.kernel,
    out_type=jax.ShapeDtypeStruct((NC, N), jnp.float32),
    mesh=_mesh,
    scratch_types=[
        [pltpu.VMEM((ET // K0_CH,), jnp.int32) for _ in range(K0_CH)],
        pltpu.VMEM((ET // K0_CH,), jnp.float32),
        pltpu.VMEM_SHARED((N,), jnp.float32),
        pltpu.SemaphoreType.DMA,
    ],
)
def _k0_degrees(idx_hbm, zeros_hbm, ones_hbm, deg_out, idx_v, ones_v, acc, sem):
    cid = lax.axis_index("c")
    sid = lax.axis_index("s")

    @pl.when(sid == 0)
    def _():
        pltpu.sync_copy(zeros_hbm, acc)

    for j in range(K0_CH):
        pltpu.sync_copy(idx_hbm.at[(cid * NS + sid) * K0_CH + j], idx_v[j])
    pltpu.sync_copy(ones_hbm, ones_v)
    plsc.subcore_barrier()

    descs = [
        pltpu.async_copy(ones_v, acc.at[idx_v[j]], sem, add=True)
        for j in range(K0_CH)
    ]
    for d in descs:
        d.wait()
    plsc.subcore_barrier()

    @pl.when(sid == 0)
    def _():
        pltpu.sync_copy(acc, deg_out.at[cid])


def _k1_body(x_ref, wg_ref, wc_ref, deg_ref, s_ref, nd_ref):
    w = jnp.dot(wg_ref[...], wc_ref[...], preferred_element_type=jnp.float32)
    t = jnp.dot(x_ref[...], w, preferred_element_type=jnp.float32)
    s_ref[...] = lax.rsqrt(jnp.maximum(deg_ref[0], 1.0)) * t.reshape(N)
    nd_ref[...] = lax.rsqrt(jnp.maximum(deg_ref[1], 1.0))


_k1_scale = pl.pallas_call(
    _k1_body,
    out_shape=(
        jax.ShapeDtypeStruct((N,), jnp.float32),
        jax.ShapeDtypeStruct((N,), jnp.float32),
    ),
)


@functools.partial(
    pl.kernel,
    out_type=jax.ShapeDtypeStruct((NC, N), jnp.float32),
    mesh=_mesh,
    scratch_types=[
        pltpu.VMEM((EW,), jnp.int32),
        [pltpu.VMEM((K2_CS,), jnp.int32) for _ in range(K2_CH)],
        pltpu.VMEM((EW,), jnp.float32),
        pltpu.VMEM_SHARED((N,), jnp.float32),
        pltpu.VMEM_SHARED((N,), jnp.float32),
        pltpu.SemaphoreType.DMA,
        pltpu.SemaphoreType.DMA,
    ],
)
def _k2_scatter(src_hbm, dst_hbm, s_hbm, zeros_hbm, parts_out,
                idx_s, idx_d, vals, acc, s_sh, gsem, ssem):
    cid = lax.axis_index("c")
    sid = lax.axis_index("s")
    wid = cid * NS + sid

    @pl.when(sid == 0)
    def _():
        pltpu.sync_copy(zeros_hbm, acc)

    @pl.when(sid == 1)
    def _():
        pltpu.sync_copy(s_hbm, s_sh)

    pltpu.sync_copy(src_hbm.at[wid], idx_s)
    for j in range(K2_CH):
        pltpu.sync_copy(dst_hbm.at[wid * K2_CH + j], idx_d[j])
    plsc.subcore_barrier()

    def _six(j):
        return idx_s.at[pl.ds(j * K2_CS, K2_CS)]

    def _vix(j):
        return vals.at[pl.ds(j * K2_CS, K2_CS)]

    gathers = [None] * K2_CH
    gathers[0] = pltpu.async_copy(s_sh.at[_six(0)], _vix(0), gsem)
    scatter = None
    for j in range(K2_CH):
        gathers[j].wait()
        if j + 1 < K2_CH:
            gathers[j + 1] = pltpu.async_copy(
                s_sh.at[_six(j + 1)], _vix(j + 1), gsem)
        if scatter is not None:
            scatter.wait()
        scatter = pltpu.async_copy(
            _vix(j), acc.at[idx_d[j]], ssem, add=True)
    scatter.wait()
    plsc.subcore_barrier()

    @pl.when(sid == 0)
    def _():
        pltpu.sync_copy(acc, parts_out.at[cid])


def _k3_body(p_ref, nd_ref, bg_ref, wc_ref, bc_ref, y_ref):
    const = jnp.sum(bg_ref[...] * wc_ref[...]) + bc_ref[0, 0]
    y = nd_ref[...] * (p_ref[0] + p_ref[1]) + const
    y_ref[...] = y.reshape(N, 1)


_k3_combine = pl.pallas_call(
    _k3_body,
    out_shape=jax.ShapeDtypeStruct((N, 1), jnp.float32),
)


def kernel(x, edge_index, W_gc, b_gc, W_cls, b_cls):
    idx_all = edge_index.reshape(NW * K0_CH, ET // K0_CH)
    zeros = jnp.zeros((N,), jnp.float32)
    ones = jnp.ones((ET // K0_CH,), jnp.float32)
    deg = _k0_degrees(idx_all, zeros, ones)

    s, nd = _k1_scale(x, W_gc, W_cls, deg)

    src_b = edge_index[0].reshape(NW, EW)
    dst_b = edge_index[1].reshape(NW * K2_CH, K2_CS)
    parts = _k2_scatter(src_b, dst_b, s, zeros)

    y = _k3_combine(parts, nd,
                    b_gc.reshape(1, D), W_cls.reshape(1, D),
                    b_cls.reshape(1, 1))
    return y
